# baseline (speedup 1.0000x reference)
import functools

import jax
import jax.numpy as jnp
from jax import lax
from jax.experimental import pallas as pl
from jax.experimental.pallas import tpu as pltpu
from jax.experimental.pallas import tpu_sc as plsc

_VOCAB = 1000000
_D = 64
_SEQ = 200
_BATCH = 4096
_NROWS = _BATCH * _SEQ
_NW = 32
_NSLICE = 4
_SLICE_SEQ = _SEQ // _NSLICE
_SLICE_ROWS = _NROWS // _NSLICE
_ROWS_PER_W = _SLICE_ROWS // _NW
_SUB = 128
_CHUNK = 256
_NSUB = _CHUNK // _SUB
_NCH = _ROWS_PER_W // _CHUNK
_NBUF = 3


def _body(x_hbm, table_hbm, out_hbm, idx_v, rows_v, sem_g):
    wid = lax.axis_index("s") * 2 + lax.axis_index("c")
    base = wid * _ROWS_PER_W
    sub0 = wid * (_ROWS_PER_W // _SUB)

    pltpu.sync_copy(x_hbm.at[pl.ds(sub0, _ROWS_PER_W // _SUB)], idx_v)

    def fire_gathers(c, buf):
        for j in range(_NSUB):
            pltpu.async_copy(
                table_hbm.at[idx_v.at[c * _NSUB + j]],
                rows_v.at[buf, pl.ds(j * _SUB, _SUB)],
                sem_g,
            )

    def wait_gathers(buf):
        pltpu.make_async_copy(
            out_hbm.at[pl.ds(0, _CHUNK)], rows_v.at[buf], sem_g
        ).wait()

    for b in range(_NBUF - 1):
        fire_gathers(b, b)

    def chunk_body(c, _):
        buf = lax.rem(c, _NBUF)
        wait_gathers(buf)

        @pl.when(c + _NBUF - 1 < _NCH)
        def _():
            fire_gathers(c + _NBUF - 1, lax.rem(c + _NBUF - 1, _NBUF))

        pltpu.sync_copy(rows_v.at[buf], out_hbm.at[pl.ds(base + c * _CHUNK, _CHUNK)])
        return 0

    lax.fori_loop(0, _NCH, chunk_body, 0)


@jax.jit
def _emb(x2, table):
    mesh = plsc.VectorSubcoreMesh(core_axis_name="c", subcore_axis_name="s")
    run = functools.partial(
        pl.kernel,
        out_type=jax.ShapeDtypeStruct((_SLICE_ROWS, _D), jnp.float32),
        mesh=mesh,
        scratch_types=[
            pltpu.VMEM((_ROWS_PER_W // _SUB, _SUB), jnp.int32),
            pltpu.VMEM((_NBUF, _CHUNK, _D), jnp.float32),
            pltpu.SemaphoreType.DMA,
        ],
        compiler_params=pltpu.CompilerParams(use_tc_tiling_on_sc=False),
    )(_body)
    return run(x2, table)


def kernel(x, token_table, pos_table):
    parts = []
    for k in range(_NSLICE):
        s0 = k * _SLICE_SEQ
        xk = x[:, s0:s0 + _SLICE_SEQ].reshape(_SLICE_ROWS // _SUB, _SUB)
        xk = xk.astype(jnp.int32)
        gk = _emb(xk, token_table)
        parts.append(
            gk.reshape(_BATCH, _SLICE_SEQ, _D)
            + pos_table[None, s0:s0 + _SLICE_SEQ, :]
        )
    return jnp.concatenate(parts, axis=1)


# speedup vs baseline: 2.1187x; 2.1187x over previous
import functools

import jax
import jax.numpy as jnp
from jax import lax
from jax.experimental import pallas as pl
from jax.experimental.pallas import tpu as pltpu
from jax.experimental.pallas import tpu_sc as plsc

_VOCAB = 1000000
_D = 64
_SEQ = 200
_BATCH = 4096
_NROWS = _BATCH * _SEQ
_NW = 32
_SLICE_ROWS = _NROWS
_ROWS_PER_W = _SLICE_ROWS // _NW
_SUB = 128
_CHUNK = 512
_NSUB = _CHUNK // _SUB
_NCH = _ROWS_PER_W // _CHUNK
_NBUF = 3


def _body(x_hbm, table_hbm, out_hbm, idx_v, rows_v, sem_g):
    wid = lax.axis_index("s") * 2 + lax.axis_index("c")
    base = wid * _ROWS_PER_W
    sub0 = wid * (_ROWS_PER_W // _SUB)

    pltpu.sync_copy(x_hbm.at[pl.ds(sub0, _ROWS_PER_W // _SUB)], idx_v)

    def fire_gathers(c, buf):
        for j in range(_NSUB):
            pltpu.async_copy(
                table_hbm.at[idx_v.at[c * _NSUB + j]],
                rows_v.at[buf, pl.ds(j * _SUB, _SUB)],
                sem_g,
            )

    def wait_gathers(buf):
        pltpu.make_async_copy(
            out_hbm.at[pl.ds(0, _CHUNK)], rows_v.at[buf], sem_g
        ).wait()

    for b in range(_NBUF - 1):
        fire_gathers(b, b)

    def chunk_body(c, _):
        buf = lax.rem(c, _NBUF)
        wait_gathers(buf)

        @pl.when(c + _NBUF - 1 < _NCH)
        def _():
            fire_gathers(c + _NBUF - 1, lax.rem(c + _NBUF - 1, _NBUF))

        pltpu.sync_copy(rows_v.at[buf], out_hbm.at[pl.ds(base + c * _CHUNK, _CHUNK)])
        return 0

    lax.fori_loop(0, _NCH, chunk_body, 0)


@jax.jit
def _emb(x2, table):
    mesh = plsc.VectorSubcoreMesh(core_axis_name="c", subcore_axis_name="s")
    run = functools.partial(
        pl.kernel,
        out_type=jax.ShapeDtypeStruct((_SLICE_ROWS, _D), jnp.float32),
        mesh=mesh,
        scratch_types=[
            pltpu.VMEM((_ROWS_PER_W // _SUB, _SUB), jnp.int32),
            pltpu.VMEM((_NBUF, _CHUNK, _D), jnp.float32),
            pltpu.SemaphoreType.DMA,
        ],
        compiler_params=pltpu.CompilerParams(use_tc_tiling_on_sc=False),
    )(_body)
    return run(x2, table)


def kernel(x, token_table, pos_table):
    x2 = x.reshape(_NROWS // _SUB, _SUB).astype(jnp.int32)
    g = _emb(x2, token_table)
    return g.reshape(_BATCH, _SEQ, _D) + pos_table[None, :, :]
